# baseline (speedup 1.0000x reference)
import functools

import jax
import jax.numpy as jnp
from jax import lax
from jax.experimental import pallas as pl
from jax.experimental.pallas import tpu as pltpu
from jax.experimental.pallas import tpu_sc as plsc

N_ATOMS = 10000
MAX_NB = 32
N_BONDS = 320000
ATOM_FDIM = 128
BOND_FDIM = 144
HID = 300

L = 16
H = 384
NV = H // L
NC, NS = 2, 16
NW = NC * NS
NA = 10240
NT = N_BONDS + NA
APW = NA // NW
BPW = N_BONDS // NW
CA = 2
CB = 40
BMA = 640

_SC_MESH = dict(core_axis_name="c", subcore_axis_name="s")


def _mm_relu_body(x_ref, w_ref, o_ref):
    o_ref[...] = jnp.maximum(
        jnp.dot(x_ref[...], w_ref[...], preferred_element_type=jnp.float32), 0.0)


def mm_relu(x, w, bm):
    m, k = x.shape
    n = w.shape[1]
    return pl.pallas_call(
        _mm_relu_body,
        grid=(m // bm,),
        in_specs=[pl.BlockSpec((bm, k), lambda i: (i, 0)),
                  pl.BlockSpec((k, n), lambda i: (0, 0))],
        out_specs=pl.BlockSpec((bm, n), lambda i: (i, 0)),
        out_shape=jax.ShapeDtypeStruct((m, n), jnp.float32),
    )(x, w)


def _mm_relu_dual_body(x_ref, w_ref, o_ref, t_ref):
    y = jnp.maximum(
        jnp.dot(x_ref[...], w_ref[...], preferred_element_type=jnp.float32), 0.0)
    o_ref[...] = y
    t_ref[...] = y


def mm_relu_dual(x, w, bm):
    m, k = x.shape
    n = w.shape[1]
    return pl.pallas_call(
        _mm_relu_dual_body,
        grid=(m // bm,),
        in_specs=[pl.BlockSpec((bm, k), lambda i: (i, 0)),
                  pl.BlockSpec((k, n), lambda i: (0, 0))],
        out_specs=[pl.BlockSpec((bm, n), lambda i: (i, 0)),
                   pl.BlockSpec((bm, n), lambda i: (i, 0))],
        out_shape=[jax.ShapeDtypeStruct((m, n), jnp.float32),
                   jax.ShapeDtypeStruct((NT, n), jnp.float32)],
    )(x, w)


def _write_atoms_body(t_ref, x_ref, o_ref):
    del t_ref
    o_ref[...] = x_ref[...]


def write_atoms(t, x):
    nblk = NA // BMA
    return pl.pallas_call(
        _write_atoms_body,
        grid=(nblk,),
        in_specs=[pl.BlockSpec(memory_space=pl.ANY),
                  pl.BlockSpec((BMA, H), lambda i: (i, 0))],
        out_specs=pl.BlockSpec((BMA, H), lambda i: (N_BONDS // BMA + i, 0)),
        out_shape=jax.ShapeDtypeStruct((NT, H), jnp.float32),
        input_output_aliases={0: 0},
    )(t, x)


def _upd_body(t_ref, agg_ref, o_ref):
    o_ref[...] = t_ref[...] + agg_ref[...]


def upd_add(t, agg):
    nblk = NA // BMA
    return pl.pallas_call(
        _upd_body,
        grid=(nblk,),
        in_specs=[pl.BlockSpec((BMA, H), lambda i: (N_BONDS // BMA + i, 0)),
                  pl.BlockSpec((BMA, H), lambda i: (i, 0))],
        out_specs=pl.BlockSpec((BMA, H), lambda i: (N_BONDS // BMA + i, 0)),
        out_shape=jax.ShapeDtypeStruct((NT, H), jnp.float32),
        input_output_aliases={0: 0},
    )(t, agg)


def _comb_body(t_ref, pre_ref, w_ref, ib_ref, o_ref):
    del t_ref
    o_ref[...] = jnp.maximum(
        ib_ref[...]
        + jnp.dot(pre_ref[...], w_ref[...], preferred_element_type=jnp.float32),
        0.0)


def comb_mm(t, pre, w, input_bond, bm):
    m = pre.shape[0]
    return pl.pallas_call(
        _comb_body,
        grid=(m // bm,),
        in_specs=[pl.BlockSpec(memory_space=pl.ANY),
                  pl.BlockSpec((bm, H), lambda i: (i, 0)),
                  pl.BlockSpec((H, H), lambda i: (0, 0)),
                  pl.BlockSpec((bm, H), lambda i: (i, 0))],
        out_specs=pl.BlockSpec((bm, H), lambda i: (i, 0)),
        out_shape=jax.ShapeDtypeStruct((NT, H), jnp.float32),
        input_output_aliases={0: 0},
    )(t, pre, w, input_bond)


def _final_body(xa_ref, xb_ref, xc_ref, wa_ref, wb_ref, wc_ref, o_ref):
    acc = jnp.dot(xa_ref[...], wa_ref[...], preferred_element_type=jnp.float32)
    acc += jnp.dot(xb_ref[...], wb_ref[...], preferred_element_type=jnp.float32)
    acc += jnp.dot(xc_ref[...], wc_ref[...], preferred_element_type=jnp.float32)
    o_ref[...] = jnp.maximum(acc, 0.0)


def final_mm(agg2, t, input_atom, wa, wb, wc):
    nblk = NA // BMA
    wspec = pl.BlockSpec((H, H), lambda i: (0, 0))
    return pl.pallas_call(
        _final_body,
        grid=(nblk,),
        in_specs=[pl.BlockSpec((BMA, H), lambda i: (i, 0)),
                  pl.BlockSpec((BMA, H), lambda i: (N_BONDS // BMA + i, 0)),
                  pl.BlockSpec((BMA, H), lambda i: (i, 0)),
                  wspec, wspec, wspec],
        out_specs=pl.BlockSpec((BMA, H), lambda i: (i, 0)),
        out_shape=jax.ShapeDtypeStruct((NA, H), jnp.float32),
    )(agg2, t, input_atom, wa, wb, wc)


def gather_reduce(t, a2b_flat):
    NCH = APW // CA
    NP = NCH // 2
    CROWS = CA * MAX_NB

    @functools.partial(
        pl.kernel,
        mesh=plsc.VectorSubcoreMesh(**_SC_MESH),
        out_type=jax.ShapeDtypeStruct((NA, H), jnp.float32),
        scratch_types=[
            pltpu.VMEM((CROWS,), jnp.int32),
            pltpu.VMEM((CROWS,), jnp.int32),
            pltpu.VMEM((CROWS, H), jnp.float32),
            pltpu.VMEM((CROWS, H), jnp.float32),
            pltpu.VMEM((CA, H), jnp.float32),
            pltpu.VMEM((CA, H), jnp.float32),
            pltpu.SemaphoreType.DMA,
            pltpu.SemaphoreType.DMA,
            pltpu.SemaphoreType.DMA,
            pltpu.SemaphoreType.DMA,
        ],
    )
    def k(t_hbm, a2b_hbm, agg_hbm, idx0, idx1, rows0, rows1, agg0, agg1,
          g0, g1, i0, i1):
        wid = lax.axis_index("s") * NC + lax.axis_index("c")
        idx = (idx0, idx1)
        rows = (rows0, rows1)
        agg = (agg0, agg1)
        gsem = (g0, g1)
        isem = (i0, i1)
        base_a = wid * APW

        def idesc(j, s):
            off = pl.multiple_of((base_a + j * CA) * MAX_NB, CROWS)
            return pltpu.make_async_copy(a2b_hbm.at[pl.ds(off, CROWS)],
                                         idx[s], isem[s])

        def gdesc(s):
            return pltpu.make_async_copy(t_hbm.at[idx[s]], rows[s], gsem[s])

        def out_sync(j, s):
            off = pl.multiple_of(base_a + j * CA, CA)
            pltpu.sync_copy(agg[s], agg_hbm.at[pl.ds(off, CA)])

        def compute(s):
            rv = rows[s]
            av = agg[s]

            def abody(a, c2):
                r0 = a * MAX_NB
                for v in range(NV):
                    sl = pl.ds(v * L, L)
                    x = rv[r0, sl]
                    sm = x
                    mx = x
                    for r in range(1, MAX_NB):
                        x = rv[r0 + r, sl]
                        sm = sm + x
                        mx = jnp.maximum(mx, x)
                    av[a, sl] = sm * mx
                return c2

            lax.fori_loop(0, CA, abody, 0)

        idesc(0, 0).start()
        idesc(1, 1).start()
        idesc(0, 0).wait()
        gdesc(0).start()
        idesc(1, 1).wait()
        gdesc(1).start()

        def body(p, carry):
            j0 = 2 * p
            j1 = j0 + 1
            gdesc(0).wait()
            idesc(j0 + 2, 0).start()
            compute(0)
            out_sync(j0, 0)
            idesc(j0 + 2, 0).wait()
            gdesc(0).start()
            gdesc(1).wait()
            idesc(j1 + 2, 1).start()
            compute(1)
            out_sync(j1, 1)
            idesc(j1 + 2, 1).wait()
            gdesc(1).start()
            return carry

        lax.fori_loop(0, NP - 1, body, 0)
        j0 = NCH - 2
        j1 = NCH - 1
        gdesc(0).wait()
        compute(0)
        out_sync(j0, 0)
        gdesc(1).wait()
        compute(1)
        out_sync(j1, 1)

    return k(t, a2b_flat)


def gather_sub(t, idx_comb):
    NCH = BPW // CB
    NP = NCH // 2
    CROWS = 2 * CB

    @functools.partial(
        pl.kernel,
        mesh=plsc.VectorSubcoreMesh(**_SC_MESH),
        out_type=jax.ShapeDtypeStruct((N_BONDS, H), jnp.float32),
        scratch_types=[
            pltpu.VMEM((CROWS,), jnp.int32),
            pltpu.VMEM((CROWS,), jnp.int32),
            pltpu.VMEM((CROWS, H), jnp.float32),
            pltpu.VMEM((CROWS, H), jnp.float32),
            pltpu.SemaphoreType.DMA,
            pltpu.SemaphoreType.DMA,
            pltpu.SemaphoreType.DMA,
            pltpu.SemaphoreType.DMA,
        ],
    )
    def k(t_hbm, idxc_hbm, pre_hbm, idx0, idx1, buf0, buf1, g0, g1, i0, i1):
        wid = lax.axis_index("s") * NC + lax.axis_index("c")
        idx = (idx0, idx1)
        buf = (buf0, buf1)
        gsem = (g0, g1)
        isem = (i0, i1)
        base_b = wid * BPW

        def idesc(j, s):
            off = pl.multiple_of((base_b + j * CB) * 2, CROWS)
            return pltpu.make_async_copy(idxc_hbm.at[pl.ds(off, CROWS)],
                                         idx[s], isem[s])

        def gdesc(s):
            return pltpu.make_async_copy(t_hbm.at[idx[s]], buf[s], gsem[s])

        def out_sync(j, s):
            off = pl.multiple_of(base_b + j * CB, CB)
            pltpu.sync_copy(buf[s].at[pl.ds(CB, CB)],
                            pre_hbm.at[pl.ds(off, CB)])

        def compute(s):
            bv = buf[s]

            def rbody(r, c2):
                for v in range(NV):
                    sl = pl.ds(v * L, L)
                    bv[CB + r, sl] = bv[CB + r, sl] - bv[r, sl]
                return c2

            lax.fori_loop(0, CB, rbody, 0)

        idesc(0, 0).start()
        idesc(1, 1).start()
        idesc(0, 0).wait()
        gdesc(0).start()
        idesc(1, 1).wait()
        gdesc(1).start()

        def body(p, carry):
            j0 = 2 * p
            j1 = j0 + 1
            gdesc(0).wait()
            idesc(j0 + 2, 0).start()
            compute(0)
            out_sync(j0, 0)
            idesc(j0 + 2, 0).wait()
            gdesc(0).start()
            gdesc(1).wait()
            idesc(j1 + 2, 1).start()
            compute(1)
            out_sync(j1, 1)
            idesc(j1 + 2, 1).wait()
            gdesc(1).start()
            return carry

        lax.fori_loop(0, NP - 1, body, 0)
        j0 = NCH - 2
        j1 = NCH - 1
        gdesc(0).wait()
        compute(0)
        out_sync(j0, 0)
        gdesc(1).wait()
        compute(1)
        out_sync(j1, 1)

    return k(t, idx_comb)


def _pad2(x, r, c):
    return jnp.pad(x, ((0, r - x.shape[0]), (0, c - x.shape[1])))


def kernel(f_atoms, f_bonds, a2b, b2a, b2revb,
           W_i_atom, W_i_bond, W_h_0, W_h_1, W_lr):
    f_atoms_p = _pad2(f_atoms, NA, ATOM_FDIM)
    wia = _pad2(W_i_atom, ATOM_FDIM, H)
    wib = _pad2(W_i_bond, BOND_FDIM, H)
    wh0 = _pad2(W_h_0, H, H)
    wh1 = _pad2(W_h_1, H, H)
    wl_a = _pad2(W_lr[0:HID], H, H)
    wl_m = _pad2(W_lr[HID:2 * HID], H, H)
    wl_i = _pad2(W_lr[2 * HID:3 * HID], H, H)

    a2b_flat = jnp.pad(a2b.astype(jnp.int32), ((0, NA - N_ATOMS), (0, 0)))
    a2b_flat = a2b_flat.reshape(-1)
    b2a32 = b2a.astype(jnp.int32)
    b2revb32 = b2revb.astype(jnp.int32)
    idx_comb = jnp.concatenate(
        [b2revb32.reshape(-1, CB), (N_BONDS + b2a32).reshape(-1, CB)],
        axis=1).reshape(-1)

    input_atom = mm_relu(f_atoms_p, wia, bm=1024)
    input_bond, t = mm_relu_dual(f_bonds, wib, bm=2000)
    t = write_atoms(t, input_atom)

    for wh in (wh0, wh1):
        agg = gather_reduce(t, a2b_flat)
        t = upd_add(t, agg)
        pre = gather_sub(t, idx_comb)
        t = comb_mm(t, pre, wh, input_bond, bm=2000)

    agg2 = gather_reduce(t, a2b_flat)
    out = final_mm(agg2, t, input_atom, wl_a, wl_m, wl_i)
    return out[1:N_ATOMS, 0:HID]


# speedup vs baseline: 1.1335x; 1.1335x over previous
import functools

import jax
import jax.numpy as jnp
from jax import lax
from jax.experimental import pallas as pl
from jax.experimental.pallas import tpu as pltpu
from jax.experimental.pallas import tpu_sc as plsc

N_ATOMS = 10000
MAX_NB = 32
N_BONDS = 320000
ATOM_FDIM = 128
BOND_FDIM = 144
HID = 300

L = 16
H = 384
NV = H // L
NC, NS = 2, 16
NW = NC * NS
NA = 10240
APW = NA // NW
BPW = N_BONDS // NW
CA = 2
CB = 80

_SC_MESH = dict(core_axis_name="c", subcore_axis_name="s")


def _mm_relu_body(x_ref, w_ref, o_ref):
    o_ref[...] = jnp.maximum(
        jnp.dot(x_ref[...], w_ref[...], preferred_element_type=jnp.float32), 0.0)


def mm_relu(x, w, bm):
    m, k = x.shape
    n = w.shape[1]
    return pl.pallas_call(
        _mm_relu_body,
        grid=(m // bm,),
        in_specs=[pl.BlockSpec((bm, k), lambda i: (i, 0)),
                  pl.BlockSpec((k, n), lambda i: (0, 0))],
        out_specs=pl.BlockSpec((bm, n), lambda i: (i, 0)),
        out_shape=jax.ShapeDtypeStruct((m, n), jnp.float32),
    )(x, w)


def _upd_body(ma_ref, agg_ref, man_ref):
    man_ref[...] = ma_ref[...] + agg_ref[...]


def upd_add(msg_atom, agg, bm):
    m = msg_atom.shape[0]
    return pl.pallas_call(
        _upd_body,
        grid=(m // bm,),
        in_specs=[pl.BlockSpec((bm, H), lambda i: (i, 0)),
                  pl.BlockSpec((bm, H), lambda i: (i, 0))],
        out_specs=pl.BlockSpec((bm, H), lambda i: (i, 0)),
        out_shape=jax.ShapeDtypeStruct((m, H), jnp.float32),
    )(msg_atom, agg)


def _comb_body(pre_ref, w_ref, ib_ref, o_ref):
    o_ref[...] = jnp.maximum(
        ib_ref[...]
        + jnp.dot(pre_ref[...], w_ref[...], preferred_element_type=jnp.float32),
        0.0)


def comb_mm(pre, w, input_bond, bm):
    m = pre.shape[0]
    return pl.pallas_call(
        _comb_body,
        grid=(m // bm,),
        in_specs=[pl.BlockSpec((bm, H), lambda i: (i, 0)),
                  pl.BlockSpec((H, H), lambda i: (0, 0)),
                  pl.BlockSpec((bm, H), lambda i: (i, 0))],
        out_specs=pl.BlockSpec((bm, H), lambda i: (i, 0)),
        out_shape=jax.ShapeDtypeStruct((m, H), jnp.float32),
    )(pre, w, input_bond)


def _final_body(xa_ref, xb_ref, xc_ref, wa_ref, wb_ref, wc_ref, o_ref):
    acc = jnp.dot(xa_ref[...], wa_ref[...], preferred_element_type=jnp.float32)
    acc += jnp.dot(xb_ref[...], wb_ref[...], preferred_element_type=jnp.float32)
    acc += jnp.dot(xc_ref[...], wc_ref[...], preferred_element_type=jnp.float32)
    o_ref[...] = jnp.maximum(acc, 0.0)


def final_mm(xa, xb, xc, wa, wb, wc, bm):
    m = xa.shape[0]
    xspec = pl.BlockSpec((bm, H), lambda i: (i, 0))
    wspec = pl.BlockSpec((H, H), lambda i: (0, 0))
    return pl.pallas_call(
        _final_body,
        grid=(m // bm,),
        in_specs=[xspec, xspec, xspec, wspec, wspec, wspec],
        out_specs=pl.BlockSpec((bm, H), lambda i: (i, 0)),
        out_shape=jax.ShapeDtypeStruct((m, H), jnp.float32),
    )(xa, xb, xc, wa, wb, wc)


def gather_reduce(msg_bond, a2b_flat):
    NCH = APW // CA
    NP = NCH // 2
    CROWS = CA * MAX_NB

    @functools.partial(
        pl.kernel,
        mesh=plsc.VectorSubcoreMesh(**_SC_MESH),
        out_type=jax.ShapeDtypeStruct((NA, H), jnp.float32),
        scratch_types=[
            pltpu.VMEM((CROWS,), jnp.int32),
            pltpu.VMEM((CROWS,), jnp.int32),
            pltpu.VMEM((CROWS, H), jnp.float32),
            pltpu.VMEM((CROWS, H), jnp.float32),
            pltpu.VMEM((2 * CA, H), jnp.float32),
            pltpu.VMEM((2 * CA, H), jnp.float32),
            pltpu.SemaphoreType.DMA,
            pltpu.SemaphoreType.DMA,
            pltpu.SemaphoreType.DMA,
            pltpu.SemaphoreType.DMA,
            pltpu.SemaphoreType.DMA,
            pltpu.SemaphoreType.DMA,
        ],
    )
    def k(msgb_hbm, a2b_hbm, agg_hbm, idx0, idx1, rows0, rows1, agg0, agg1,
          g0, g1, o0, o1, i0, i1):
        wid = lax.axis_index("s") * NC + lax.axis_index("c")
        idx = (idx0, idx1)
        rows = (rows0, rows1)
        agg = (agg0, agg1)
        gsem = (g0, g1)
        osem = (o0, o1)
        isem = (i0, i1)
        base_a = wid * APW

        def idesc(j, s):
            off = pl.multiple_of((base_a + j * CA) * MAX_NB, CROWS)
            return pltpu.make_async_copy(a2b_hbm.at[pl.ds(off, CROWS)],
                                         idx[s], isem[s])

        def gdesc(s):
            return pltpu.make_async_copy(msgb_hbm.at[idx[s]], rows[s],
                                         gsem[s])

        def out_sync4(j0):
            off = pl.multiple_of(base_a + j0 * CA, 2 * CA)
            pltpu.sync_copy(agg0, agg_hbm.at[pl.ds(off, 2 * CA)])

        def compute(s, ro):
            rv = rows[s]

            def abody(a, c2):
                r0 = a * MAX_NB
                for v in range(NV):
                    sl = pl.ds(v * L, L)
                    x = rv[r0, sl]
                    sm = x
                    mx = x
                    for r in range(1, MAX_NB):
                        x = rv[r0 + r, sl]
                        sm = sm + x
                        mx = jnp.maximum(mx, x)
                    agg0[ro + a, sl] = sm * mx
                return c2

            lax.fori_loop(0, CA, abody, 0)

        idesc(0, 0).start()
        idesc(1, 1).start()
        idesc(0, 0).wait()
        gdesc(0).start()
        idesc(1, 1).wait()
        gdesc(1).start()

        def body(p, carry):
            j0 = 2 * p
            j1 = j0 + 1
            gdesc(0).wait()
            idesc(j0 + 2, 0).start()
            compute(0, 0)
            idesc(j0 + 2, 0).wait()
            gdesc(0).start()
            gdesc(1).wait()
            idesc(j1 + 2, 1).start()
            compute(1, CA)
            out_sync4(j0)
            idesc(j1 + 2, 1).wait()
            gdesc(1).start()
            return carry

        lax.fori_loop(0, NP - 1, body, 0)
        j0 = NCH - 2
        gdesc(0).wait()
        compute(0, 0)
        gdesc(1).wait()
        compute(1, CA)
        out_sync4(j0)

    return k(msg_bond, a2b_flat)


def gather_sub(a2t, msg_bond, b2a, b2revb):
    NCH = BPW // CB
    NP = NCH // 2

    @functools.partial(
        pl.kernel,
        mesh=plsc.VectorSubcoreMesh(**_SC_MESH),
        out_type=jax.ShapeDtypeStruct((N_BONDS, H), jnp.float32),
        scratch_types=[
            pltpu.VMEM((CB,), jnp.int32),
            pltpu.VMEM((CB,), jnp.int32),
            pltpu.VMEM((CB,), jnp.int32),
            pltpu.VMEM((CB,), jnp.int32),
            pltpu.VMEM((CB, H), jnp.float32),
            pltpu.VMEM((CB, H), jnp.float32),
            pltpu.VMEM((CB, H), jnp.float32),
            pltpu.VMEM((CB, H), jnp.float32),
            pltpu.SemaphoreType.DMA,
            pltpu.SemaphoreType.DMA,
            pltpu.SemaphoreType.DMA,
            pltpu.SemaphoreType.DMA,
            pltpu.SemaphoreType.DMA,
            pltpu.SemaphoreType.DMA,
        ],
    )
    def k(a2_hbm, msgb_hbm, b2a_hbm, b2revb_hbm, pre_hbm,
          idxa0, idxa1, idxr0, idxr1, bufa0, bufa1, bufr0, bufr1,
          g0, g1, o0, o1, i0, i1):
        wid = lax.axis_index("s") * NC + lax.axis_index("c")
        idxa = (idxa0, idxa1)
        idxr = (idxr0, idxr1)
        iasem = (i0, i1)
        irsem = (o0, o1)
        del bufa1, bufr1, g1
        base_b = pl.multiple_of(wid * BPW, CB)

        def ia_desc(j, s):
            off = pl.multiple_of(base_b + j * CB, CB)
            return pltpu.make_async_copy(b2a_hbm.at[pl.ds(off, CB)],
                                         idxa[s], iasem[s])

        def ir_desc(j, s):
            off = pl.multiple_of(base_b + j * CB, CB)
            return pltpu.make_async_copy(b2revb_hbm.at[pl.ds(off, CB)],
                                         idxr[s], irsem[s])

        def istart(j, s):
            ia_desc(j, s).start()
            ir_desc(j, s).start()

        def iwait(j, s):
            ia_desc(j, s).wait()
            ir_desc(j, s).wait()

        def compute():
            def rbody(r, c2):
                for v in range(NV):
                    sl = pl.ds(v * L, L)
                    bufa0[r, sl] = bufa0[r, sl] - bufr0[r, sl]
                return c2

            lax.fori_loop(0, CB, rbody, 0)

        def chunk(j, s, prefetch):
            b0 = pl.multiple_of(base_b + j * CB, CB)
            iwait(j, s)
            pltpu.async_copy(a2_hbm.at[idxa[s]], bufa0, g0).wait()
            pltpu.async_copy(msgb_hbm.at[idxr[s]], bufr0, g0).wait()
            if prefetch:
                istart(j + 2, s)
            compute()
            pltpu.sync_copy(bufa0, pre_hbm.at[pl.ds(b0, CB)])

        istart(0, 0)
        istart(1, 1)

        def body(p, carry):
            j0 = 2 * p
            chunk(j0, 0, True)
            chunk(j0 + 1, 1, True)
            return carry

        lax.fori_loop(0, NP - 1, body, 0)
        chunk(NCH - 3, 0, True)
        chunk(NCH - 2, 1, False)
        chunk(NCH - 1, 0, False)

    return k(a2t, msg_bond, b2a, b2revb)


def _pad2(x, r, c):
    return jnp.pad(x, ((0, r - x.shape[0]), (0, c - x.shape[1])))


def kernel(f_atoms, f_bonds, a2b, b2a, b2revb,
           W_i_atom, W_i_bond, W_h_0, W_h_1, W_lr):
    f_atoms_p = _pad2(f_atoms, NA, ATOM_FDIM)
    wia = _pad2(W_i_atom, ATOM_FDIM, H)
    wib = _pad2(W_i_bond, BOND_FDIM, H)
    wh0 = _pad2(W_h_0, H, H)
    wh1 = _pad2(W_h_1, H, H)
    wl_a = _pad2(W_lr[0:HID], H, H)
    wl_m = _pad2(W_lr[HID:2 * HID], H, H)
    wl_i = _pad2(W_lr[2 * HID:3 * HID], H, H)

    a2b_flat = jnp.pad(a2b.astype(jnp.int32), ((0, NA - N_ATOMS), (0, 0)))
    a2b_flat = a2b_flat.reshape(-1)
    b2a32 = b2a.astype(jnp.int32)
    b2revb32 = b2revb.astype(jnp.int32)

    input_atom = mm_relu(f_atoms_p, wia, bm=1024)
    input_bond = mm_relu(f_bonds, wib, bm=2000)

    msg_atom = input_atom
    msg_bond = input_bond
    for wh in (wh0, wh1):
        agg = gather_reduce(msg_bond, a2b_flat)
        msg_atom = upd_add(msg_atom, agg, bm=1024)
        pre = gather_sub(msg_atom, msg_bond, b2a32, b2revb32)
        msg_bond = comb_mm(pre, wh, input_bond, bm=2000)

    agg2 = gather_reduce(msg_bond, a2b_flat)
    out = final_mm(agg2, msg_atom, input_atom, wl_a, wl_m, wl_i, bm=1024)
    return out[1:N_ATOMS, 0:HID]
